# Initial kernel scaffold; baseline (speedup 1.0000x reference)
import jax
import jax.numpy as jnp
from jax.experimental import pallas as pl


def kernel(wav):
    raise NotImplementedError("write your pallas kernel here")


# speedup vs baseline: 4.0927x; 4.0927x over previous
import functools

import jax
import jax.numpy as jnp
import numpy as np
from jax import lax
from jax.experimental import pallas as pl
from jax.experimental.pallas import tpu as pltpu
from jax.experimental.pallas import tpu_sc as plsc

SHIFT = 8192
TIME = 262144
LENGTH = TIME - SHIFT
BATCH, SRCS, CHANS = 16, 4, 2
ROWS = BATCH * SRCS * CHANS

_OFFS = np.array([
    7932,  943,  736, 7017, 6560, 4780, 3350, 7573,
    5970, 2818, 6176, 4161, 2562, 1378, 8174, 6170,
    4241, 7767,   91, 4026, 6840, 5652,   56, 3018,
    7654, 6943, 5978, 2339, 2760, 4806, 2916, 6151,
    2835, 1851, 5685, 1105, 2937, 6570, 3980, 7714,
    1846, 3866, 2914, 1936, 4871, 2834, 6125, 3618,
    7487,  593, 7351, 5479,  572, 1765, 3913, 2687,
    5257, 6147, 7228, 2263, 3132,  722, 1196, 1763,
], dtype=np.int32)

NUM_CORES = 2
NUM_SUBCORES = 16
NW = NUM_CORES * NUM_SUBCORES
ROWS_PER_W = ROWS // NW
CHUNK = 31744
NCH = LENGTH // CHUNK
UNROLL = 8


def _row_offset(pair):
    off = jnp.int32(_OFFS[0])
    for p in range(1, 64):
        off = lax.select(pair == p, jnp.int32(int(_OFFS[p])), off)
    return off


def _sc_shift(wav2):
    mesh = plsc.VectorSubcoreMesh(
        core_axis_name="c", subcore_axis_name="s",
        num_cores=NUM_CORES, num_subcores=NUM_SUBCORES)

    @functools.partial(
        pl.kernel,
        out_type=jax.ShapeDtypeStruct((ROWS, LENGTH), jnp.float32),
        mesh=mesh,
        scratch_types=(
            [pltpu.VMEM((CHUNK + 8,), jnp.float32) for _ in range(2)]
            + [pltpu.VMEM((CHUNK,), jnp.float32) for _ in range(2)]
            + [pltpu.SemaphoreType.DMA for _ in range(4)]
        ),
        compiler_params=pltpu.CompilerParams(use_tc_tiling_on_sc=False),
    )
    def k(wav_hbm, out_hbm, ib0, ib1, ob0, ob1, si0, si1, so0, so1):
        ibufs, obufs = [ib0, ib1], [ob0, ob1]
        isems, osems = [si0, si1], [so0, so1]
        wid = lax.axis_index("s") * NUM_CORES + lax.axis_index("c")
        row0 = wid * ROWS_PER_W
        offs = [_row_offset(wid * 2), _row_offset(wid * 2 + 1)]
        als = [pl.multiple_of((o >> 3) << 3, 8) for o in offs]
        rs = [offs[j] - als[j] for j in range(2)]

        units = [(i, c) for i in range(ROWS_PER_W) for c in range(NCH)]
        NU = len(units)

        def in_copy(u):
            i, c = units[u]
            start = pl.multiple_of(als[i // 2] + c * CHUNK, 8)
            src = wav_hbm.at[row0 + i, pl.ds(start, CHUNK + 8)]
            return pltpu.make_async_copy(src, ibufs[u % 2], isems[u % 2])

        def out_copy(u):
            i, c = units[u]
            dst = out_hbm.at[row0 + i, pl.ds(c * CHUNK, CHUNK)]
            return pltpu.make_async_copy(obufs[u % 2], dst, osems[u % 2])

        def shift(u):
            i, c = units[u]
            r = rs[i // 2]
            ib, ob = ibufs[u % 2], obufs[u % 2]

            def body(kk, _):
                b = kk * (16 * UNROLL)
                for j in range(UNROLL):
                    o = pl.multiple_of(b + j * 16, 16)
                    ob[pl.ds(o, 16)] = ib[pl.ds(r + o, 16)]
                return 0

            lax.fori_loop(0, CHUNK // (16 * UNROLL), body, 0)

        in_copy(0).start()
        in_copy(1).start()
        for u in range(NU):
            in_copy(u).wait()
            if u >= 2:
                out_copy(u - 2).wait()
            shift(u)
            out_copy(u).start()
            if u + 2 < NU:
                in_copy(u + 2).start()
        out_copy(NU - 2).wait()
        out_copy(NU - 1).wait()

    return k(wav2)


def kernel(wav):
    wav2 = wav.reshape(ROWS, TIME)
    out2 = _sc_shift(wav2)
    return out2.reshape(BATCH, SRCS, CHANS, LENGTH)


# speedup vs baseline: 4.1026x; 1.0024x over previous
import functools

import jax
import jax.numpy as jnp
import numpy as np
from jax import lax
from jax.experimental import pallas as pl
from jax.experimental.pallas import tpu as pltpu
from jax.experimental.pallas import tpu_sc as plsc

SHIFT = 8192
TIME = 262144
LENGTH = TIME - SHIFT
BATCH, SRCS, CHANS = 16, 4, 2
ROWS = BATCH * SRCS * CHANS

_OFFS = np.array([
    7932,  943,  736, 7017, 6560, 4780, 3350, 7573,
    5970, 2818, 6176, 4161, 2562, 1378, 8174, 6170,
    4241, 7767,   91, 4026, 6840, 5652,   56, 3018,
    7654, 6943, 5978, 2339, 2760, 4806, 2916, 6151,
    2835, 1851, 5685, 1105, 2937, 6570, 3980, 7714,
    1846, 3866, 2914, 1936, 4871, 2834, 6125, 3618,
    7487,  593, 7351, 5479,  572, 1765, 3913, 2687,
    5257, 6147, 7228, 2263, 3132,  722, 1196, 1763,
], dtype=np.int32)

NUM_CORES = 2
NUM_SUBCORES = 16
NW = NUM_CORES * NUM_SUBCORES
ROWS_PER_W = ROWS // NW
CHUNK = 31744
NCH = LENGTH // CHUNK
UNROLL = 16


def _row_offset(pair):
    off = jnp.int32(_OFFS[0])
    for p in range(1, 64):
        off = lax.select(pair == p, jnp.int32(int(_OFFS[p])), off)
    return off


def _sc_shift(wav2):
    mesh = plsc.VectorSubcoreMesh(
        core_axis_name="c", subcore_axis_name="s",
        num_cores=NUM_CORES, num_subcores=NUM_SUBCORES)

    @functools.partial(
        pl.kernel,
        out_type=jax.ShapeDtypeStruct((ROWS, LENGTH), jnp.float32),
        mesh=mesh,
        scratch_types=(
            [pltpu.VMEM((CHUNK + 8,), jnp.float32) for _ in range(2)]
            + [pltpu.VMEM((CHUNK,), jnp.float32) for _ in range(2)]
            + [pltpu.SemaphoreType.DMA for _ in range(4)]
        ),
        compiler_params=pltpu.CompilerParams(use_tc_tiling_on_sc=False),
    )
    def k(wav_hbm, out_hbm, ib0, ib1, ob0, ob1, si0, si1, so0, so1):
        ibufs, obufs = [ib0, ib1], [ob0, ob1]
        isems, osems = [si0, si1], [so0, so1]
        wid = lax.axis_index("s") * NUM_CORES + lax.axis_index("c")
        row0 = wid * ROWS_PER_W
        offs = [_row_offset(wid * 2), _row_offset(wid * 2 + 1)]
        als = [pl.multiple_of((o >> 3) << 3, 8) for o in offs]
        rs = [offs[j] - als[j] for j in range(2)]

        units = [(i, c) for i in range(ROWS_PER_W) for c in range(NCH)]
        NU = len(units)

        def in_copy(u):
            i, c = units[u]
            start = pl.multiple_of(als[i // 2] + c * CHUNK, 8)
            src = wav_hbm.at[row0 + i, pl.ds(start, CHUNK + 8)]
            return pltpu.make_async_copy(src, ibufs[u % 2], isems[u % 2])

        def out_copy(u):
            i, c = units[u]
            dst = out_hbm.at[row0 + i, pl.ds(c * CHUNK, CHUNK)]
            return pltpu.make_async_copy(obufs[u % 2], dst, osems[u % 2])

        def shift(u):
            i, c = units[u]
            r = rs[i // 2]
            ib, ob = ibufs[u % 2], obufs[u % 2]

            def body(kk, _):
                b = kk * (16 * UNROLL)
                for j in range(UNROLL):
                    o = pl.multiple_of(b + j * 16, 16)
                    ob[pl.ds(o, 16)] = ib[pl.ds(r + o, 16)]
                return 0

            lax.fori_loop(0, CHUNK // (16 * UNROLL), body, 0)

        in_copy(0).start()
        in_copy(1).start()
        for u in range(NU):
            in_copy(u).wait()
            if u >= 2:
                out_copy(u - 2).wait()
            shift(u)
            out_copy(u).start()
            if u + 2 < NU:
                in_copy(u + 2).start()
        out_copy(NU - 2).wait()
        out_copy(NU - 1).wait()

    return k(wav2)


def kernel(wav):
    wav2 = wav.reshape(ROWS, TIME)
    out2 = _sc_shift(wav2)
    return out2.reshape(BATCH, SRCS, CHANS, LENGTH)


# speedup vs baseline: 9.4531x; 2.3041x over previous
import functools

import jax
import jax.numpy as jnp
import numpy as np
from jax import lax
from jax.experimental import pallas as pl
from jax.experimental.pallas import tpu as pltpu
from jax.experimental.pallas import tpu_sc as plsc

SHIFT = 8192
TIME = 262144
LENGTH = TIME - SHIFT
BATCH, SRCS, CHANS = 16, 4, 2
PAIRS = BATCH * SRCS
TB = TIME // 128
LB = LENGTH // 128

_OFFS = np.array([
    7932,  943,  736, 7017, 6560, 4780, 3350, 7573,
    5970, 2818, 6176, 4161, 2562, 1378, 8174, 6170,
    4241, 7767,   91, 4026, 6840, 5652,   56, 3018,
    7654, 6943, 5978, 2339, 2760, 4806, 2916, 6151,
    2835, 1851, 5685, 1105, 2937, 6570, 3980, 7714,
    1846, 3866, 2914, 1936, 4871, 2834, 6125, 3618,
    7487,  593, 7351, 5479,  572, 1765, 3913, 2687,
    5257, 6147, 7228, 2263, 3132,  722, 1196, 1763,
], dtype=np.int32)

NUM_CORES = 2
NUM_SUBCORES = 16
NW = NUM_CORES * NUM_SUBCORES
PAIRS_PER_W = PAIRS // NW
CB = 124
NCH = LB // CB
IN_W = (CB + 1) * 256
OUT_W = CB * 256


def _pair_offset(pair):
    off = jnp.int32(_OFFS[0])
    for p in range(1, PAIRS):
        off = lax.select(pair == p, jnp.int32(int(_OFFS[p])), off)
    return off


def _sc_shift(wav_p):
    mesh = plsc.VectorSubcoreMesh(
        core_axis_name="c", subcore_axis_name="s",
        num_cores=NUM_CORES, num_subcores=NUM_SUBCORES)

    @functools.partial(
        pl.kernel,
        out_type=jax.ShapeDtypeStruct((PAIRS * LB * 256,), jnp.float32),
        mesh=mesh,
        scratch_types=(
            [pltpu.VMEM((IN_W,), jnp.float32) for _ in range(2)]
            + [pltpu.VMEM((OUT_W,), jnp.float32) for _ in range(2)]
            + [pltpu.SemaphoreType.DMA for _ in range(4)]
        ),
        compiler_params=pltpu.CompilerParams(
            use_tc_tiling_on_sc=False, needs_layout_passes=False),
    )
    def k(wav_hbm, out_hbm, ib0, ib1, ob0, ob1, si0, si1, so0, so1):
        ibufs, obufs = [ib0, ib1], [ob0, ob1]
        isems, osems = [si0, si1], [so0, so1]
        wid = lax.axis_index("s") * NUM_CORES + lax.axis_index("c")

        pairs = [wid * PAIRS_PER_W + j for j in range(PAIRS_PER_W)]
        offs = [_pair_offset(p) for p in pairs]
        qs = [o >> 7 for o in offs]
        rs = [o & 127 for o in offs]

        iota = lax.iota(jnp.int32, 16)

        units = [(j, c) for j in range(PAIRS_PER_W) for c in range(NCH)]
        NU = len(units)

        def in_copy(u):
            j, c = units[u]
            start = pl.multiple_of((pairs[j] * TB + qs[j] + c * CB) * 256, 256)
            src = wav_hbm.at[pl.ds(start, IN_W)]
            return pltpu.make_async_copy(src, ibufs[u % 2], isems[u % 2])

        def out_copy(u):
            j, c = units[u]
            start = pl.multiple_of((pairs[j] * LB + c * CB) * 256, 256)
            dst = out_hbm.at[pl.ds(start, OUT_W)]
            return pltpu.make_async_copy(obufs[u % 2], dst, osems[u % 2])

        def realign(u):
            j, _ = units[u]
            r = rs[j]
            ib, ob = ibufs[u % 2], obufs[u % 2]
            ks = []
            for ch in range(CHANS):
                for w0 in range(0, 128, 16):
                    t = r + w0 + iota
                    ks.append((t >> 7) * 256 + (ch * 128 + (t & 127)))

            def body(blk, _):
                base = blk * 256
                basev = jnp.full((16,), 0, jnp.int32) + base
                for i in range(16):
                    v = plsc.load_gather(ib, [ks[i] + basev])
                    o = pl.multiple_of(base + i * 16, 16)
                    ob[pl.ds(o, 16)] = v
                return 0

            lax.fori_loop(0, CB, body, 0)

        in_copy(0).start()
        in_copy(1).start()
        for u in range(NU):
            in_copy(u).wait()
            if u >= 2:
                out_copy(u - 2).wait()
            realign(u)
            out_copy(u).start()
            if u + 2 < NU:
                in_copy(u + 2).start()
        out_copy(NU - 2).wait()
        out_copy(NU - 1).wait()

    return k(wav_p)


def kernel(wav):
    wav_p = wav.reshape(BATCH, SRCS, CHANS, TB, 128)
    wav_p = wav_p.transpose(0, 1, 3, 2, 4).reshape(-1)
    out_p = _sc_shift(wav_p)
    out = out_p.reshape(BATCH, SRCS, LB, CHANS, 128)
    out = out.transpose(0, 1, 3, 2, 4).reshape(BATCH, SRCS, CHANS, LENGTH)
    return out


# speedup vs baseline: 22.6814x; 2.3994x over previous
import functools

import jax
import jax.numpy as jnp
import numpy as np
from jax import lax
from jax.experimental import pallas as pl
from jax.experimental.pallas import tpu as pltpu
from jax.experimental.pallas import tpu_sc as plsc

SHIFT = 8192
TIME = 262144
LENGTH = TIME - SHIFT
BATCH, SRCS, CHANS = 16, 4, 2
PAIRS = BATCH * SRCS
TB = TIME // 128
LB = LENGTH // 128

_OFFS = np.array([
    7932,  943,  736, 7017, 6560, 4780, 3350, 7573,
    5970, 2818, 6176, 4161, 2562, 1378, 8174, 6170,
    4241, 7767,   91, 4026, 6840, 5652,   56, 3018,
    7654, 6943, 5978, 2339, 2760, 4806, 2916, 6151,
    2835, 1851, 5685, 1105, 2937, 6570, 3980, 7714,
    1846, 3866, 2914, 1936, 4871, 2834, 6125, 3618,
    7487,  593, 7351, 5479,  572, 1765, 3913, 2687,
    5257, 6147, 7228, 2263, 3132,  722, 1196, 1763,
], dtype=np.int32)

NUM_CORES = 2
NUM_SUBCORES = 16
NW = NUM_CORES * NUM_SUBCORES
PAIRS_PER_W = PAIRS // NW
CB = 124
NCH = LB // CB
IN_W = (CB + 1) * 256
OUT_W = CB * 256


def _pair_offset(pair):
    off = jnp.int32(_OFFS[0])
    for p in range(1, PAIRS):
        off = lax.select(pair == p, jnp.int32(int(_OFFS[p])), off)
    return off


def _sc_shift(wav_p):
    mesh = plsc.VectorSubcoreMesh(
        core_axis_name="c", subcore_axis_name="s",
        num_cores=NUM_CORES, num_subcores=NUM_SUBCORES)

    @functools.partial(
        pl.kernel,
        out_type=jax.ShapeDtypeStruct((PAIRS * LB * 256,), jnp.float32),
        mesh=mesh,
        scratch_types=(
            [pltpu.VMEM((IN_W,), jnp.float32) for _ in range(2)]
            + [pltpu.VMEM((OUT_W,), jnp.float32) for _ in range(2)]
            + [pltpu.SemaphoreType.DMA for _ in range(4)]
        ),
        compiler_params=pltpu.CompilerParams(
            use_tc_tiling_on_sc=False, needs_layout_passes=False),
    )
    def k(wav_hbm, out_hbm, ib0, ib1, ob0, ob1, si0, si1, so0, so1):
        ibufs, obufs = [ib0, ib1], [ob0, ob1]
        isems, osems = [si0, si1], [so0, so1]
        wid = lax.axis_index("s") * NUM_CORES + lax.axis_index("c")

        pairs = [wid * PAIRS_PER_W + j for j in range(PAIRS_PER_W)]
        offs = [_pair_offset(p) for p in pairs]
        qs = [o >> 7 for o in offs]
        rs = [o & 127 for o in offs]

        iota = lax.iota(jnp.int32, 16)

        units = [(j, c) for j in range(PAIRS_PER_W) for c in range(NCH)]
        NU = len(units)

        def in_copy(u):
            j, c = units[u]
            start = pl.multiple_of((pairs[j] * TB + qs[j] + c * CB) * 256, 256)
            src = wav_hbm.at[pl.ds(start, IN_W)]
            return pltpu.make_async_copy(src, ibufs[u % 2], isems[u % 2])

        def out_copy(u):
            j, c = units[u]
            start = pl.multiple_of((pairs[j] * LB + c * CB) * 256, 256)
            dst = out_hbm.at[pl.ds(start, OUT_W)]
            return pltpu.make_async_copy(obufs[u % 2], dst, osems[u % 2])

        def realign(u):
            j, _ = units[u]
            r = rs[j]
            ib, ob = ibufs[u % 2], obufs[u % 2]
            ks = []
            for ch in range(CHANS):
                for w0 in range(0, 128, 16):
                    t = r + w0 + iota
                    ks.append((t >> 7) * 256 + (ch * 128 + (t & 127)))

            @plsc.parallel_loop(0, CB, 1, unroll=1)
            def body(blk):
                base = blk * 256
                basev = jnp.full((16,), 0, jnp.int32) + base
                for i in range(16):
                    v = plsc.load_gather(ib, [ks[i] + basev])
                    o = pl.multiple_of(base + i * 16, 16)
                    ob[pl.ds(o, 16)] = v

        in_copy(0).start()
        in_copy(1).start()
        for u in range(NU):
            in_copy(u).wait()
            if u >= 2:
                out_copy(u - 2).wait()
            realign(u)
            out_copy(u).start()
            if u + 2 < NU:
                in_copy(u + 2).start()
        out_copy(NU - 2).wait()
        out_copy(NU - 1).wait()

    return k(wav_p)


def kernel(wav):
    wav_p = wav.reshape(BATCH, SRCS, CHANS, TB, 128)
    wav_p = wav_p.transpose(0, 1, 3, 2, 4).reshape(-1)
    out_p = _sc_shift(wav_p)
    out = out_p.reshape(BATCH, SRCS, LB, CHANS, 128)
    out = out.transpose(0, 1, 3, 2, 4).reshape(BATCH, SRCS, CHANS, LENGTH)
    return out


# speedup vs baseline: 23.0700x; 1.0171x over previous
import functools

import jax
import jax.numpy as jnp
import numpy as np
from jax import lax
from jax.experimental import pallas as pl
from jax.experimental.pallas import tpu as pltpu
from jax.experimental.pallas import tpu_sc as plsc

SHIFT = 8192
TIME = 262144
LENGTH = TIME - SHIFT
BATCH, SRCS, CHANS = 16, 4, 2
PAIRS = BATCH * SRCS
TB = TIME // 128
LB = LENGTH // 128

_OFFS = np.array([
    7932,  943,  736, 7017, 6560, 4780, 3350, 7573,
    5970, 2818, 6176, 4161, 2562, 1378, 8174, 6170,
    4241, 7767,   91, 4026, 6840, 5652,   56, 3018,
    7654, 6943, 5978, 2339, 2760, 4806, 2916, 6151,
    2835, 1851, 5685, 1105, 2937, 6570, 3980, 7714,
    1846, 3866, 2914, 1936, 4871, 2834, 6125, 3618,
    7487,  593, 7351, 5479,  572, 1765, 3913, 2687,
    5257, 6147, 7228, 2263, 3132,  722, 1196, 1763,
], dtype=np.int32)

NUM_CORES = 2
NUM_SUBCORES = 16
NW = NUM_CORES * NUM_SUBCORES
PAIRS_PER_W = PAIRS // NW
CB = 62
NCH = LB // CB
IN_W = (CB + 1) * 256
OUT_W = CB * 256
NB = 4
NG = NCH // NB


def _pair_offset(pair):
    off = jnp.int32(_OFFS[0])
    for p in range(1, PAIRS):
        off = lax.select(pair == p, jnp.int32(int(_OFFS[p])), off)
    return off


def _sc_shift(wav_p):
    mesh = plsc.VectorSubcoreMesh(
        core_axis_name="c", subcore_axis_name="s",
        num_cores=NUM_CORES, num_subcores=NUM_SUBCORES)

    @functools.partial(
        pl.kernel,
        out_type=jax.ShapeDtypeStruct((PAIRS * LB * 256,), jnp.float32),
        mesh=mesh,
        scratch_types=(
            [pltpu.VMEM((IN_W,), jnp.float32) for _ in range(NB)]
            + [pltpu.VMEM((OUT_W,), jnp.float32) for _ in range(NB)]
            + [pltpu.SemaphoreType.DMA for _ in range(2 * NB)]
        ),
        compiler_params=pltpu.CompilerParams(
            use_tc_tiling_on_sc=False, needs_layout_passes=False),
    )
    def k(wav_hbm, out_hbm, *scr):
        ibufs, obufs = list(scr[:NB]), list(scr[NB:2 * NB])
        isems, osems = list(scr[2 * NB:3 * NB]), list(scr[3 * NB:])
        wid = lax.axis_index("s") * NUM_CORES + lax.axis_index("c")

        pairs = [wid * PAIRS_PER_W + j for j in range(PAIRS_PER_W)]
        offs = [_pair_offset(p) for p in pairs]
        qs = [o >> 7 for o in offs]
        rs = [o & 127 for o in offs]

        iota = lax.iota(jnp.int32, 16)

        for j in range(PAIRS_PER_W):
            in_base = (pairs[j] * TB + qs[j]) * 256
            out_base = pairs[j] * LB * 256
            r = rs[j]

            def in_copy(c, b, in_base=in_base):
                start = pl.multiple_of(in_base + c * (CB * 256), 256)
                src = wav_hbm.at[pl.ds(start, IN_W)]
                return pltpu.make_async_copy(src, ibufs[b], isems[b])

            def out_copy(c, b, out_base=out_base):
                start = pl.multiple_of(out_base + c * (CB * 256), 256)
                dst = out_hbm.at[pl.ds(start, OUT_W)]
                return pltpu.make_async_copy(obufs[b], dst, osems[b])

            ks = []
            for ch in range(CHANS):
                for w0 in range(0, 128, 16):
                    t = r + w0 + iota
                    ks.append((t >> 7) * 256 + (ch * 128 + (t & 127)))

            def realign(b, ks=ks):
                ib, ob = ibufs[b], obufs[b]

                @plsc.parallel_loop(0, CB, 1, unroll=1)
                def body(blk):
                    base = blk * 256
                    basev = jnp.full((16,), 0, jnp.int32) + base
                    for i in range(16):
                        v = plsc.load_gather(ib, [ks[i] + basev])
                        o = pl.multiple_of(base + i * 16, 16)
                        ob[pl.ds(o, 16)] = v

            for b in range(NB):
                in_copy(jnp.int32(b), b).start()
            for b in range(NB):
                in_copy(jnp.int32(b), b).wait()
                realign(b)
                out_copy(jnp.int32(b), b).start()
                in_copy(jnp.int32(NB + b), b).start()

            def group(g, _):
                for b in range(NB):
                    c = g * NB + b
                    in_copy(c, b).wait()
                    out_copy(c - NB, b).wait()
                    realign(b)
                    out_copy(c, b).start()
                    in_copy(c + NB, b).start()
                return 0

            lax.fori_loop(1, NG - 1, group, 0)

            for b in range(NB):
                c = jnp.int32((NG - 1) * NB + b)
                in_copy(c, b).wait()
                out_copy(c - NB, b).wait()
                realign(b)
                out_copy(c, b).start()
            for b in range(NB):
                out_copy(jnp.int32((NG - 1) * NB + b), b).wait()

    return k(wav_p)


def kernel(wav):
    wav_p = wav.reshape(BATCH, SRCS, CHANS, TB, 128)
    wav_p = wav_p.transpose(0, 1, 3, 2, 4).reshape(-1)
    out_p = _sc_shift(wav_p)
    out = out_p.reshape(BATCH, SRCS, LB, CHANS, 128)
    out = out.transpose(0, 1, 3, 2, 4).reshape(BATCH, SRCS, CHANS, LENGTH)
    return out


# speedup vs baseline: 24.0185x; 1.0411x over previous
import functools

import jax
import jax.numpy as jnp
import numpy as np
from jax import lax
from jax.experimental import pallas as pl
from jax.experimental.pallas import tpu as pltpu
from jax.experimental.pallas import tpu_sc as plsc

SHIFT = 8192
TIME = 262144
LENGTH = TIME - SHIFT
BATCH, SRCS, CHANS = 16, 4, 2
PAIRS = BATCH * SRCS
TB = TIME // 128
LB = LENGTH // 128

_OFFS = np.array([
    7932,  943,  736, 7017, 6560, 4780, 3350, 7573,
    5970, 2818, 6176, 4161, 2562, 1378, 8174, 6170,
    4241, 7767,   91, 4026, 6840, 5652,   56, 3018,
    7654, 6943, 5978, 2339, 2760, 4806, 2916, 6151,
    2835, 1851, 5685, 1105, 2937, 6570, 3980, 7714,
    1846, 3866, 2914, 1936, 4871, 2834, 6125, 3618,
    7487,  593, 7351, 5479,  572, 1765, 3913, 2687,
    5257, 6147, 7228, 2263, 3132,  722, 1196, 1763,
], dtype=np.int32)

NUM_CORES = 2
NUM_SUBCORES = 16
NW = NUM_CORES * NUM_SUBCORES
PAIRS_PER_W = PAIRS // NW
CB = 62
NCH = LB // CB
IN_W = (CB + 1) * 256
OUT_W = CB * 256
NB = 4
NU = PAIRS_PER_W * NCH
NG = NU // NB


def _pair_offset(pair):
    off = jnp.int32(_OFFS[0])
    for p in range(1, PAIRS):
        off = lax.select(pair == p, jnp.int32(int(_OFFS[p])), off)
    return off


def _sc_shift(wav_p):
    mesh = plsc.VectorSubcoreMesh(
        core_axis_name="c", subcore_axis_name="s",
        num_cores=NUM_CORES, num_subcores=NUM_SUBCORES)

    @functools.partial(
        pl.kernel,
        out_type=jax.ShapeDtypeStruct((PAIRS * LB * 256,), jnp.float32),
        mesh=mesh,
        scratch_types=(
            [pltpu.VMEM((IN_W,), jnp.float32) for _ in range(NB)]
            + [pltpu.VMEM((OUT_W,), jnp.float32) for _ in range(NB)]
            + [pltpu.SemaphoreType.DMA for _ in range(2 * NB)]
        ),
        compiler_params=pltpu.CompilerParams(
            use_tc_tiling_on_sc=False, needs_layout_passes=False),
    )
    def k(wav_hbm, out_hbm, *scr):
        ibufs, obufs = list(scr[:NB]), list(scr[NB:2 * NB])
        isems, osems = list(scr[2 * NB:3 * NB]), list(scr[3 * NB:])
        wid = lax.axis_index("s") * NUM_CORES + lax.axis_index("c")

        pairs = [wid * PAIRS_PER_W + j for j in range(PAIRS_PER_W)]
        offs = [_pair_offset(p) for p in pairs]
        in_bases = [(pairs[j] * TB + (offs[j] >> 7)) * 256
                    for j in range(PAIRS_PER_W)]
        out_bases = [pairs[j] * LB * 256 for j in range(PAIRS_PER_W)]
        rs = [o & 127 for o in offs]

        iota = lax.iota(jnp.int32, 16)

        kss = []
        for j in range(PAIRS_PER_W):
            ks = []
            for ch in range(CHANS):
                for w0 in range(0, 128, 16):
                    t = rs[j] + w0 + iota
                    ks.append((t >> 7) * 256 + (ch * 128 + (t & 127)))
            kss.append(ks)

        def in_copy(u, b):
            start = pl.multiple_of(
                in_bases[b % 2] + (u // 2) * (CB * 256), 256)
            src = wav_hbm.at[pl.ds(start, IN_W)]
            return pltpu.make_async_copy(src, ibufs[b], isems[b])

        def out_copy(u, b):
            start = pl.multiple_of(
                out_bases[b % 2] + (u // 2) * (CB * 256), 256)
            dst = out_hbm.at[pl.ds(start, OUT_W)]
            return pltpu.make_async_copy(obufs[b], dst, osems[b])

        def realign(b):
            ib, ob = ibufs[b], obufs[b]
            ks = kss[b % 2]

            @plsc.parallel_loop(0, CB, 1, unroll=1)
            def body(blk):
                base = blk * 256
                basev = jnp.full((16,), 0, jnp.int32) + base
                for i in range(16):
                    v = plsc.load_gather(ib, [ks[i] + basev])
                    o = pl.multiple_of(base + i * 16, 16)
                    ob[pl.ds(o, 16)] = v

        for b in range(NB):
            in_copy(jnp.int32(b), b).start()
        for b in range(NB):
            in_copy(jnp.int32(b), b).wait()
            realign(b)
            out_copy(jnp.int32(b), b).start()
            in_copy(jnp.int32(NB + b), b).start()

        def group(g, _):
            for b in range(NB):
                u = g * NB + b
                in_copy(u, b).wait()
                out_copy(u - NB, b).wait()
                realign(b)
                out_copy(u, b).start()
                in_copy(u + NB, b).start()
            return 0

        lax.fori_loop(1, NG - 1, group, 0)

        for b in range(NB):
            u = jnp.int32((NG - 1) * NB + b)
            in_copy(u, b).wait()
            out_copy(u - NB, b).wait()
            realign(b)
            out_copy(u, b).start()
        for b in range(NB):
            out_copy(jnp.int32((NG - 1) * NB + b), b).wait()

    return k(wav_p)


def kernel(wav):
    wav_p = wav.reshape(BATCH, SRCS, CHANS, TB, 128)
    wav_p = wav_p.transpose(0, 1, 3, 2, 4).reshape(-1)
    out_p = _sc_shift(wav_p)
    out = out_p.reshape(BATCH, SRCS, LB, CHANS, 128)
    out = out.transpose(0, 1, 3, 2, 4).reshape(BATCH, SRCS, CHANS, LENGTH)
    return out
